# Initial kernel scaffold; baseline (speedup 1.0000x reference)
import jax
import jax.numpy as jnp
from jax.experimental import pallas as pl


def kernel(inputs, idx, emb0, emb1, emb2):
    raise NotImplementedError("write your pallas kernel here")


# speedup vs baseline: 25.2190x; 25.2190x over previous
import jax
import jax.numpy as jnp
from jax.experimental import pallas as pl
from jax.experimental.pallas import tpu as pltpu

K = 2048
D = 64
NIMG = 32
PIX = 1024
PB = 512
GRID = NIMG * PIX // PB
NTOT = NIMG * PIX

_PREC = jax.lax.Precision.DEFAULT


def _body(x_ref, codes_ref, codesT_ref, q_ref, loss_ref, perp_ref, counts_ref):
    g = pl.program_id(0)
    xb = x_ref[0]
    codes = codes_ref[...]
    c2 = jnp.sum(codes * codes, axis=1, keepdims=True)
    scores = c2 - 2.0 * jax.lax.dot(codes, xb, precision=_PREC)
    m = jnp.min(scores, axis=0, keepdims=True)
    iota0 = jax.lax.broadcasted_iota(jnp.int32, (K, PB), 0)
    idxm = jnp.where(scores == m, iota0, K)
    enc = jnp.min(idxm, axis=0, keepdims=True)
    oh = (iota0 == enc).astype(jnp.float32)
    q_ref[0] = jax.lax.dot(codesT_ref[...], oh, precision=_PREC)

    x2 = jnp.sum(xb * xb, axis=0, keepdims=True)
    step_loss = jnp.sum(m + x2, axis=1, keepdims=True)
    cnt = jnp.sum(oh, axis=1, keepdims=True)

    @pl.when(g == 0)
    def _():
        counts_ref[...] = cnt
        loss_ref[...] = step_loss

    @pl.when(g > 0)
    def _():
        counts_ref[...] += cnt
        loss_ref[...] += step_loss

    @pl.when(g == GRID - 1)
    def _():
        avg = counts_ref[...] * (1.0 / NTOT)
        ent = jnp.sum(avg * jnp.log(avg + 1e-10), axis=0, keepdims=True)
        perp_ref[...] = jnp.exp(-ent)
        loss_ref[...] = loss_ref[...] * (1.25 / (NTOT * D))


def kernel(inputs, idx, emb0, emb1, emb2):
    x = inputs.reshape(NIMG, D, PIX)
    codes = jnp.concatenate([emb0, jnp.where(idx == 1, emb1, emb2)], axis=0)
    codesT = codes.T

    q, loss, perp = pl.pallas_call(
        _body,
        grid=(GRID,),
        in_specs=[
            pl.BlockSpec((1, D, PB), lambda g: (g // (PIX // PB), 0, g % (PIX // PB))),
            pl.BlockSpec((K, D), lambda g: (0, 0)),
            pl.BlockSpec((D, K), lambda g: (0, 0)),
        ],
        out_specs=[
            pl.BlockSpec((1, D, PB), lambda g: (g // (PIX // PB), 0, g % (PIX // PB))),
            pl.BlockSpec((1, 1), lambda g: (0, 0)),
            pl.BlockSpec((1, 1), lambda g: (0, 0)),
        ],
        out_shape=[
            jax.ShapeDtypeStruct((NIMG, D, PIX), jnp.float32),
            jax.ShapeDtypeStruct((1, 1), jnp.float32),
            jax.ShapeDtypeStruct((1, 1), jnp.float32),
        ],
        scratch_shapes=[pltpu.VMEM((K, 1), jnp.float32)],
    )(x, codes, codesT)

    return q.reshape(NIMG, D, 32, 32), loss[0, 0], perp[0, 0]


# speedup vs baseline: 26.5414x; 1.0524x over previous
import jax
import jax.numpy as jnp
from jax.experimental import pallas as pl
from jax.experimental.pallas import tpu as pltpu

K = 2048
D = 64
DA = 72
NIMG = 32
PIX = 1024
PB = 512
GRID = NIMG * PIX // PB
NTOT = NIMG * PIX

_PREC = jax.lax.Precision.DEFAULT


def _body(x_ref, caug_ref, codesT_ref, q_ref, loss_ref, perp_ref, counts_ref):
    g = pl.program_id(0)
    xb = x_ref[0]
    x_aug = jnp.concatenate(
        [xb, jnp.ones((DA - D, PB), jnp.float32)], axis=0)
    scores = jax.lax.dot(caug_ref[...], x_aug, precision=_PREC)
    m = jnp.min(scores, axis=0, keepdims=True)
    eq = (scores == m).astype(jnp.float32)
    t = jnp.sum(eq, axis=0, keepdims=True)
    q_ref[0] = jax.lax.dot(codesT_ref[...], eq, precision=_PREC) / t

    x2 = jnp.sum(xb * xb, axis=0, keepdims=True)
    step_loss = jnp.sum(m + x2, axis=1, keepdims=True)
    cnt8 = jax.lax.dot(eq, jnp.ones((PB, 8), jnp.float32),
                       precision=_PREC)

    @pl.when(g == 0)
    def _():
        counts_ref[...] = cnt8
        loss_ref[...] = step_loss

    @pl.when(g > 0)
    def _():
        counts_ref[...] += cnt8
        loss_ref[...] += step_loss

    @pl.when(g == GRID - 1)
    def _():
        avg = counts_ref[:, :1] * (1.0 / NTOT)
        ent = jnp.sum(avg * jnp.log(avg + 1e-10), axis=0, keepdims=True)
        perp_ref[...] = jnp.exp(-ent)
        loss_ref[...] = loss_ref[...] * (1.25 / (NTOT * D))


def kernel(inputs, idx, emb0, emb1, emb2):
    x = inputs.reshape(NIMG, D, PIX)
    codes = jnp.concatenate([emb0, jnp.where(idx == 1, emb1, emb2)], axis=0)
    c2 = jnp.sum(codes * codes, axis=1, keepdims=True)
    caug = jnp.concatenate(
        [codes * -2.0, c2, jnp.zeros((K, DA - D - 1), jnp.float32)], axis=1)
    codesT = codes.T

    q, loss, perp = pl.pallas_call(
        _body,
        grid=(GRID,),
        in_specs=[
            pl.BlockSpec((1, D, PB), lambda g: (g // (PIX // PB), 0, g % (PIX // PB))),
            pl.BlockSpec((K, DA), lambda g: (0, 0)),
            pl.BlockSpec((D, K), lambda g: (0, 0)),
        ],
        out_specs=[
            pl.BlockSpec((1, D, PB), lambda g: (g // (PIX // PB), 0, g % (PIX // PB))),
            pl.BlockSpec((1, 1), lambda g: (0, 0)),
            pl.BlockSpec((1, 1), lambda g: (0, 0)),
        ],
        out_shape=[
            jax.ShapeDtypeStruct((NIMG, D, PIX), jnp.float32),
            jax.ShapeDtypeStruct((1, 1), jnp.float32),
            jax.ShapeDtypeStruct((1, 1), jnp.float32),
        ],
        scratch_shapes=[pltpu.VMEM((K, 8), jnp.float32)],
    )(x, caug, codesT)

    return q.reshape(NIMG, D, 32, 32), loss[0, 0], perp[0, 0]
